# baseline (speedup 1.0000x reference)
import functools

import jax
import jax.numpy as jnp
import numpy as np
from jax import lax
from jax.experimental import pallas as pl
from jax.experimental.pallas import tpu as pltpu
from jax.experimental.pallas import tpu_sc as plsc

_FIELD_DIMS = [100000] * 26
_NF = len(_FIELD_DIMS)
_B = 16384
_V = 2600000
_L = 16
_NC, _NS = 2, 16
_NW = _NC * _NS
_BPW = _B // _NW
_GPW = _BPW * _NF
_CHUNKS = _BPW // _L

_OFFSETS = np.concatenate(([0], np.cumsum(np.array(_FIELD_DIMS))[:-1])).astype(np.int32)

_mesh = plsc.VectorSubcoreMesh(core_axis_name="c", subcore_axis_name="s")


_FCHUNK = 163840
_FBASES = [(b, min(_FCHUNK, _V - b)) for b in range(0, _V, _FCHUNK)]


def _flatten_body(t_ref, o_ref, sem):
    cps = [
        pltpu.make_async_copy(
            t_ref.at[0, pl.ds(b, n)], o_ref.at[pl.ds(b, n)], sem
        )
        for b, n in _FBASES
    ]
    for cp in cps:
        cp.start()
    for cp in cps:
        cp.wait()


_tc_flatten_call = pl.pallas_call(
    _flatten_body,
    in_specs=[pl.BlockSpec(memory_space=pltpu.MemorySpace.HBM)],
    out_specs=pl.BlockSpec(memory_space=pltpu.MemorySpace.HBM),
    out_shape=jax.ShapeDtypeStruct((_V,), jnp.float32),
    scratch_shapes=[pltpu.SemaphoreType.DMA],
)


def _tc_flatten(table):
    return _tc_flatten_call(lax.transpose(table, (1, 0)))


@functools.partial(
    pl.kernel,
    out_type=jax.ShapeDtypeStruct((_B,), jnp.float32),
    mesh=_mesh,
    scratch_types=[
        pltpu.VMEM((_GPW,), jnp.int32),
        pltpu.VMEM((_GPW,), jnp.float32),
        pltpu.VMEM((_BPW,), jnp.float32),
        pltpu.SemaphoreType.DMA,
    ],
    compiler_params=pltpu.CompilerParams(needs_layout_passes=False),
)
def _sc_lookup(idx_hbm, table_hbm, out_hbm, idx_v, rows_v, out_v, sem):
    wid = lax.axis_index("s") * _NC + lax.axis_index("c")
    gbase = wid * _GPW
    obase = wid * _BPW

    pltpu.sync_copy(idx_hbm.at[pl.ds(gbase, _GPW)], idx_v)
    pltpu.async_copy(table_hbm.at[idx_v], rows_v, sem).wait()

    lanes = lax.iota(jnp.int32, _L) * _NF

    def chunk_body(c, _):
        base = c * (_L * _NF)
        acc = jnp.zeros((_L,), jnp.float32)
        for f in range(_NF):
            acc = acc + plsc.load_gather(rows_v, [lanes + (base + f)])
        out_v[pl.ds(c * _L, _L)] = acc
        return _

    lax.fori_loop(0, _CHUNKS, chunk_body, None)
    pltpu.sync_copy(out_v, out_hbm.at[pl.ds(obase, _BPW)])


def kernel(x, table, bias):
    offsets = jnp.asarray(_OFFSETS)
    idx = (x + offsets[None, :]).reshape(-1)
    wx = _sc_lookup(idx, _tc_flatten(table))
    return wx[:, None] + bias[None, :]


# speedup vs baseline: 5.5608x; 5.5608x over previous
import functools

import jax
import jax.numpy as jnp
import numpy as np
from jax import lax
from jax.experimental import pallas as pl
from jax.experimental.pallas import tpu as pltpu
from jax.experimental.pallas import tpu_sc as plsc

_FIELD_DIMS = [100000] * 26
_NF = len(_FIELD_DIMS)
_B = 16384
_V = 2600000
_L = 16
_NC, _NS = 2, 16
_NW = _NC * _NS
_BPW = _B // _NW
_GPW = _BPW * _NF
_CHUNKS = _BPW // _L

_OFFSETS = np.concatenate(([0], np.cumsum(np.array(_FIELD_DIMS))[:-1])).astype(np.int32)

_mesh = plsc.VectorSubcoreMesh(core_axis_name="c", subcore_axis_name="s")


_FBLK = 163840
_FGRID = (_V + _FBLK - 1) // _FBLK


def _flatten_body(t_ref, o_ref):
    o_ref[...] = t_ref[0, :]


_tc_flatten_call = pl.pallas_call(
    _flatten_body,
    grid=(_FGRID,),
    in_specs=[pl.BlockSpec((1, _FBLK), lambda i: (0, i))],
    out_specs=pl.BlockSpec((_FBLK,), lambda i: (i,)),
    out_shape=jax.ShapeDtypeStruct((_V,), jnp.float32),
)


def _tc_flatten(table):
    return _tc_flatten_call(lax.transpose(table, (1, 0)))


@functools.partial(
    pl.kernel,
    out_type=jax.ShapeDtypeStruct((_B,), jnp.float32),
    mesh=_mesh,
    scratch_types=[
        pltpu.VMEM((_GPW,), jnp.int32),
        pltpu.VMEM((_GPW,), jnp.float32),
        pltpu.VMEM((_BPW,), jnp.float32),
        pltpu.SemaphoreType.DMA,
    ],
    compiler_params=pltpu.CompilerParams(needs_layout_passes=False),
)
def _sc_lookup(idx_hbm, table_hbm, out_hbm, idx_v, rows_v, out_v, sem):
    wid = lax.axis_index("s") * _NC + lax.axis_index("c")
    gbase = wid * _GPW
    obase = wid * _BPW

    pltpu.sync_copy(idx_hbm.at[pl.ds(gbase, _GPW)], idx_v)
    pltpu.async_copy(table_hbm.at[idx_v], rows_v, sem).wait()

    lanes = lax.iota(jnp.int32, _L) * _NF

    def chunk_body(c, _):
        base = c * (_L * _NF)
        acc = jnp.zeros((_L,), jnp.float32)
        for f in range(_NF):
            acc = acc + plsc.load_gather(rows_v, [lanes + (base + f)])
        out_v[pl.ds(c * _L, _L)] = acc
        return _

    lax.fori_loop(0, _CHUNKS, chunk_body, None)
    pltpu.sync_copy(out_v, out_hbm.at[pl.ds(obase, _BPW)])


def kernel(x, table, bias):
    offsets = jnp.asarray(_OFFSETS)
    idx = (x + offsets[None, :]).reshape(-1)
    wx = _sc_lookup(idx, _tc_flatten(table))
    return wx[:, None] + bias[None, :]


# speedup vs baseline: 6.9284x; 1.2459x over previous
import functools

import jax
import jax.numpy as jnp
import numpy as np
from jax import lax
from jax.experimental import pallas as pl
from jax.experimental.pallas import tpu as pltpu
from jax.experimental.pallas import tpu_sc as plsc

_FIELD_DIMS = [100000] * 26
_NF = len(_FIELD_DIMS)
_B = 16384
_V = 2600000
_L = 16
_NC, _NS = 2, 16
_NW = _NC * _NS
_BPW = _B // _NW
_GPW = _BPW * _NF
_CHUNKS = _BPW // _L

_OFFSETS = np.concatenate(([0], np.cumsum(np.array(_FIELD_DIMS))[:-1])).astype(np.int32)

_mesh = plsc.VectorSubcoreMesh(core_axis_name="c", subcore_axis_name="s")

_FBLK = 163840
_FGRID = (_V + _FBLK - 1) // _FBLK


def _flatten_body(t_ref, o_ref):
    o_ref[...] = t_ref[0, :]


_tc_flatten_call = pl.pallas_call(
    _flatten_body,
    grid=(_FGRID,),
    in_specs=[pl.BlockSpec((1, _FBLK), lambda i: (0, i))],
    out_specs=pl.BlockSpec((_FBLK,), lambda i: (i,)),
    out_shape=jax.ShapeDtypeStruct((_V,), jnp.float32),
)


def _tc_flatten(table):
    return _tc_flatten_call(lax.transpose(table, (1, 0)))


@functools.partial(
    pl.kernel,
    out_type=jax.ShapeDtypeStruct((_B,), jnp.float32),
    mesh=_mesh,
    scratch_types=[
        pltpu.VMEM((_NF, _BPW), jnp.int32),
        pltpu.VMEM((_GPW,), jnp.int32),
        pltpu.VMEM((_GPW,), jnp.float32),
        pltpu.VMEM((_BPW,), jnp.float32),
        pltpu.VMEM((8,), jnp.float32),
        pltpu.SemaphoreType.DMA,
    ],
    compiler_params=pltpu.CompilerParams(needs_layout_passes=False),
)
def _sc_lookup(xt_hbm, table_hbm, bias_hbm, out_hbm, xv, idx_v, rows_v, out_v, bias_s, sem):
    wid = lax.axis_index("s") * _NC + lax.axis_index("c")
    b0 = wid * _BPW

    pltpu.sync_copy(xt_hbm.at[:, pl.ds(b0, _BPW)], xv)
    pltpu.sync_copy(bias_hbm, bias_s.at[pl.ds(0, 1)])

    def idx_body(c, _):
        for f in range(_NF):
            idx_v[pl.ds(f * _BPW + c * _L, _L)] = xv[f, pl.ds(c * _L, _L)] + _OFFSETS[f]
        return _

    lax.fori_loop(0, _CHUNKS, idx_body, None)

    cps = [
        pltpu.async_copy(
            table_hbm.at[idx_v.at[pl.ds(f * _BPW, _BPW)]],
            rows_v.at[pl.ds(f * _BPW, _BPW)],
            sem,
        )
        for f in range(_NF)
    ]
    for cp in cps:
        cp.wait()

    bias = plsc.load_gather(bias_s, [jnp.zeros((_L,), jnp.int32)])

    def chunk_body(c, _):
        acc = bias
        for f in range(_NF):
            acc = acc + rows_v[pl.ds(f * _BPW + c * _L, _L)]
        out_v[pl.ds(c * _L, _L)] = acc
        return _

    lax.fori_loop(0, _CHUNKS, chunk_body, None)
    pltpu.sync_copy(out_v, out_hbm.at[pl.ds(b0, _BPW)])


def kernel(x, table, bias):
    wx = _sc_lookup(lax.transpose(x, (1, 0)), _tc_flatten(table), bias)
    return wx[:, None]


# speedup vs baseline: 7.4695x; 1.0781x over previous
import functools

import jax
import jax.numpy as jnp
import numpy as np
from jax import lax
from jax.experimental import pallas as pl
from jax.experimental.pallas import tpu as pltpu
from jax.experimental.pallas import tpu_sc as plsc

_FIELD_DIMS = [100000] * 26
_NF = len(_FIELD_DIMS)
_B = 16384
_V = 2600000
_L = 16
_NC, _NS = 2, 16
_NW = _NC * _NS
_BPW = _B // _NW
_GPW = _BPW * _NF
_CHUNKS = _BPW // _L

_OFFSETS = np.concatenate(([0], np.cumsum(np.array(_FIELD_DIMS))[:-1])).astype(np.int32)

_mesh = plsc.VectorSubcoreMesh(core_axis_name="c", subcore_axis_name="s")

_FBLK = 163840
_FGRID = (_V + _FBLK - 1) // _FBLK


def _flatten_body(t_ref, o_ref):
    o_ref[...] = t_ref[0, :]


_tc_flatten_call = pl.pallas_call(
    _flatten_body,
    grid=(_FGRID,),
    in_specs=[pl.BlockSpec((1, _FBLK), lambda i: (0, i))],
    out_specs=pl.BlockSpec((_FBLK,), lambda i: (i,)),
    out_shape=jax.ShapeDtypeStruct((_V,), jnp.float32),
)


def _tc_flatten(table):
    return _tc_flatten_call(lax.transpose(table, (1, 0)))


@functools.partial(
    pl.kernel,
    out_type=jax.ShapeDtypeStruct((_B,), jnp.float32),
    mesh=_mesh,
    scratch_types=[
        pltpu.VMEM((_NF, _BPW), jnp.int32),
        pltpu.VMEM((_GPW,), jnp.int32),
        pltpu.VMEM((_GPW,), jnp.float32),
        pltpu.VMEM((_BPW,), jnp.float32),
        pltpu.VMEM((8,), jnp.float32),
        pltpu.SemaphoreType.DMA((_NF,)),
    ],
    compiler_params=pltpu.CompilerParams(needs_layout_passes=False),
)
def _sc_lookup(xt_hbm, table_hbm, bias_hbm, out_hbm, xv, idx_v, rows_v, out_v, bias_s, sems):
    wid = lax.axis_index("s") * _NC + lax.axis_index("c")
    b0 = wid * _BPW

    pltpu.sync_copy(xt_hbm.at[:, pl.ds(b0, _BPW)], xv)
    pltpu.sync_copy(bias_hbm, bias_s.at[pl.ds(0, 1)])

    cps = []
    for f in range(_NF):
        def idx_body(c, _, f=f):
            idx_v[pl.ds(f * _BPW + c * _L, _L)] = xv[f, pl.ds(c * _L, _L)] + _OFFSETS[f]
            return _

        lax.fori_loop(0, _CHUNKS, idx_body, None)
        cps.append(
            pltpu.async_copy(
                table_hbm.at[idx_v.at[pl.ds(f * _BPW, _BPW)]],
                rows_v.at[pl.ds(f * _BPW, _BPW)],
                sems.at[f],
            )
        )

    bias = plsc.load_gather(bias_s, [jnp.zeros((_L,), jnp.int32)])

    def seed_body(c, _):
        out_v[pl.ds(c * _L, _L)] = bias
        return _

    lax.fori_loop(0, _CHUNKS, seed_body, None)

    for f in range(_NF):
        cps[f].wait()

        def acc_body(c, _, f=f):
            plsc.addupdate(
                out_v.at[pl.ds(c * _L, _L)],
                rows_v[pl.ds(f * _BPW + c * _L, _L)],
            )
            return _

        lax.fori_loop(0, _CHUNKS, acc_body, None)

    pltpu.sync_copy(out_v, out_hbm.at[pl.ds(b0, _BPW)])


def kernel(x, table, bias):
    wx = _sc_lookup(lax.transpose(x, (1, 0)), _tc_flatten(table), bias)
    return wx[:, None]


# speedup vs baseline: 8.1857x; 1.0959x over previous
import functools

import jax
import jax.numpy as jnp
import numpy as np
from jax import lax
from jax.experimental import pallas as pl
from jax.experimental.pallas import tpu as pltpu
from jax.experimental.pallas import tpu_sc as plsc

_FIELD_DIMS = [100000] * 26
_NF = len(_FIELD_DIMS)
_B = 16384
_V = 2600000
_L = 16
_NC, _NS = 2, 16
_NW = _NC * _NS
_BPW = _B // _NW
_GPW = _BPW * _NF
_CHUNKS = _BPW // _L

_OFFSETS = np.concatenate(([0], np.cumsum(np.array(_FIELD_DIMS))[:-1])).astype(np.int32)

_mesh = plsc.VectorSubcoreMesh(core_axis_name="c", subcore_axis_name="s")

_FBLK = 327680
_FGRID = (_V + _FBLK - 1) // _FBLK


def _flatten_body(t_ref, o_ref):
    o_ref[...] = t_ref[0, :]


_tc_flatten_call = pl.pallas_call(
    _flatten_body,
    grid=(_FGRID,),
    in_specs=[pl.BlockSpec((1, _FBLK), lambda i: (0, i))],
    out_specs=pl.BlockSpec((_FBLK,), lambda i: (i,)),
    out_shape=jax.ShapeDtypeStruct((_V,), jnp.float32),
)


def _tc_flatten(table):
    return _tc_flatten_call(lax.transpose(table, (1, 0)))


@functools.partial(
    pl.kernel,
    out_type=jax.ShapeDtypeStruct((_B,), jnp.float32),
    mesh=_mesh,
    scratch_types=[
        pltpu.VMEM((_NF, _BPW), jnp.int32),
        pltpu.VMEM((_GPW,), jnp.int32),
        pltpu.VMEM((_GPW,), jnp.float32),
        pltpu.VMEM((_BPW,), jnp.float32),
        pltpu.VMEM((8,), jnp.float32),
        pltpu.SemaphoreType.DMA((_NF,)),
    ],
    compiler_params=pltpu.CompilerParams(needs_layout_passes=False),
)
def _sc_lookup(xt_hbm, table_hbm, bias_hbm, out_hbm, xv, idx_v, rows_v, out_v, bias_s, sems):
    wid = lax.axis_index("s") * _NC + lax.axis_index("c")
    b0 = wid * _BPW

    pltpu.sync_copy(xt_hbm.at[:, pl.ds(b0, _BPW)], xv)
    pltpu.sync_copy(bias_hbm, bias_s.at[pl.ds(0, 1)])

    cps = []
    for f in range(_NF):
        def idx_body(c, _, f=f):
            idx_v[pl.ds(f * _BPW + c * _L, _L)] = xv[f, pl.ds(c * _L, _L)] + _OFFSETS[f]
            return _

        lax.fori_loop(0, _CHUNKS, idx_body, None)
        cps.append(
            pltpu.async_copy(
                table_hbm.at[idx_v.at[pl.ds(f * _BPW, _BPW)]],
                rows_v.at[pl.ds(f * _BPW, _BPW)],
                sems.at[f],
            )
        )

    bias = plsc.load_gather(bias_s, [jnp.zeros((_L,), jnp.int32)])

    def seed_body(c, _):
        out_v[pl.ds(c * _L, _L)] = bias
        return _

    lax.fori_loop(0, _CHUNKS, seed_body, None)

    for f in range(_NF):
        cps[f].wait()

        def acc_body(c, _, f=f):
            plsc.addupdate(
                out_v.at[pl.ds(c * _L, _L)],
                rows_v[pl.ds(f * _BPW + c * _L, _L)],
            )
            return _

        lax.fori_loop(0, _CHUNKS, acc_body, None)

    pltpu.sync_copy(out_v, out_hbm.at[pl.ds(b0, _BPW)])


def kernel(x, table, bias):
    wx = _sc_lookup(lax.transpose(x, (1, 0)), _tc_flatten(table), bias)
    return wx[:, None]
